# speedup vs baseline: 5.1828x; 5.1828x over previous
import functools

import jax
import jax.numpy as jnp
from jax import lax
from jax.experimental import pallas as pl
from jax.experimental.pallas import tpu as pltpu
from jax.experimental.pallas import tpu_sc as plsc

_NUM_CORES = 2
_NUM_SUBCORES = 16
_NUM_WORKERS = _NUM_CORES * _NUM_SUBCORES

_CHUNK = 128


@functools.partial(jax.jit, static_argnums=(2, 3))
def _embedding_gather(table, idx3, n_chunks, embed_dim):
  total = _NUM_WORKERS * n_chunks * _CHUNK
  mesh = plsc.VectorSubcoreMesh(
      core_axis_name="c", subcore_axis_name="s", num_cores=_NUM_CORES)

  @functools.partial(
      pl.kernel,
      mesh=mesh,
      out_type=jax.ShapeDtypeStruct((total, embed_dim), jnp.float32),
      scratch_types=[
          pltpu.VMEM((2, _CHUNK), jnp.int32),
          pltpu.VMEM((2, _CHUNK, embed_dim), jnp.float32),
          pltpu.SemaphoreType.DMA,
          pltpu.SemaphoreType.DMA,
      ],
  )
  def k(table_hbm, idx_hbm, out_hbm, idx_v, rows_v, gsem, osem):
    wid = lax.axis_index("s") * _NUM_CORES + lax.axis_index("c")

    def chunk(j, slot):
      pltpu.sync_copy(idx_hbm.at[wid, j], idx_v.at[slot])
      pltpu.async_copy(table_hbm.at[idx_v.at[slot]], rows_v.at[slot],
                       gsem).wait()
      base = (wid * n_chunks + j) * _CHUNK
      pltpu.sync_copy(rows_v.at[slot], out_hbm.at[pl.ds(base, _CHUNK)])
      return 0

    lax.fori_loop(0, n_chunks, chunk, 0)

  return k(table, idx3)


def kernel(input_x, table):
  batch, hist = input_x.shape
  _, embed_dim = table.shape
  total = batch * hist
  assert total % (_NUM_WORKERS * _CHUNK) == 0
  n_chunks = total // (_NUM_WORKERS * _CHUNK)
  idx3 = input_x.astype(jnp.int32).reshape(_NUM_WORKERS, n_chunks, _CHUNK)
  out = _embedding_gather(table, idx3, n_chunks, embed_dim)
  return out.reshape(batch, hist, embed_dim)


# speedup vs baseline: 9.2150x; 1.7780x over previous
import functools

import jax
import jax.numpy as jnp
from jax import lax
from jax.experimental import pallas as pl
from jax.experimental.pallas import tpu as pltpu
from jax.experimental.pallas import tpu_sc as plsc

_NUM_CORES = 2
_NUM_SUBCORES = 16
_NUM_WORKERS = _NUM_CORES * _NUM_SUBCORES

_CHUNK = 128
_K = 2


@functools.partial(jax.jit, static_argnums=(2, 3))
def _embedding_gather(table, idx3, n_chunks, embed_dim):
  total = _NUM_WORKERS * n_chunks * _CHUNK
  group = _K * _CHUNK
  n_groups = n_chunks // _K
  assert n_chunks % _K == 0 and n_groups % 2 == 0 and n_groups >= 4
  mesh = plsc.VectorSubcoreMesh(
      core_axis_name="c", subcore_axis_name="s", num_cores=_NUM_CORES)

  @functools.partial(
      pl.kernel,
      mesh=mesh,
      out_type=jax.ShapeDtypeStruct((total, embed_dim), jnp.float32),
      scratch_types=[
          pltpu.VMEM((n_chunks, _CHUNK), jnp.int32),
          pltpu.VMEM((2, group, embed_dim), jnp.float32),
          pltpu.SemaphoreType.DMA,
          pltpu.SemaphoreType.DMA,
          pltpu.SemaphoreType.DMA,
          pltpu.SemaphoreType.DMA,
      ],
  )
  def k(table_hbm, idx_hbm, out_hbm, idx_all, rows_v, gsem0, gsem1, osem0,
        osem1):
    wid = lax.axis_index("s") * _NUM_CORES + lax.axis_index("c")
    pltpu.sync_copy(idx_hbm.at[wid], idx_all)
    out_base = wid * n_chunks * _CHUNK
    gsems = (gsem0, gsem1)
    osems = (osem0, osem1)

    def fire_group(g, par):
      for b in range(_K):
        pltpu.async_copy(
            table_hbm.at[idx_all.at[g * _K + b]],
            rows_v.at[par, pl.ds(b * _CHUNK, _CHUNK)],
            gsems[par])

    def wait_gather(par):
      pltpu.make_async_copy(
          table_hbm.at[pl.ds(0, group)], rows_v.at[par], gsems[par]).wait()

    def fire_out(g, par):
      pltpu.async_copy(
          rows_v.at[par], out_hbm.at[pl.ds(out_base + g * group, group)],
          osems[par])

    def wait_out(par):
      pltpu.make_async_copy(
          rows_v.at[par], out_hbm.at[pl.ds(0, group)], osems[par]).wait()

    fire_group(0, 0)
    fire_group(1, 1)
    wait_gather(0)
    fire_out(0, 0)

    def pair(h, _):
      g1 = 2 * h + 1
      wait_out(0)
      fire_group(g1 + 1, 0)
      wait_gather(1)
      fire_out(g1, 1)
      wait_out(1)
      fire_group(g1 + 2, 1)
      wait_gather(0)
      fire_out(g1 + 1, 0)
      return 0

    lax.fori_loop(0, n_groups // 2 - 1, pair, 0)

    wait_out(0)
    wait_gather(1)
    fire_out(n_groups - 1, 1)
    wait_out(1)

  return k(table, idx3)


def kernel(input_x, table):
  batch, hist = input_x.shape
  _, embed_dim = table.shape
  total = batch * hist
  assert total % (_NUM_WORKERS * _CHUNK) == 0
  n_chunks = total // (_NUM_WORKERS * _CHUNK)
  idx3 = input_x.astype(jnp.int32).reshape(_NUM_WORKERS, n_chunks, _CHUNK)
  out = _embedding_gather(table, idx3, n_chunks, embed_dim)
  return out.reshape(batch, hist, embed_dim)
